# baseline (speedup 1.0000x reference)
import functools

import numpy as np
import jax
import jax.numpy as jnp
from jax.experimental import pallas as pl

_NBINS = 25
_EPS = 1e-6
_TAPS = np.array([1.0, 6.0, 15.0, 20.0, 15.0, 6.0, 1.0], dtype=np.float64) / 64.0


def _blur_matrix(n: int) -> np.ndarray:
    m = n // 2
    a = np.zeros((n, m), dtype=np.float64)
    for j in range(m):
        for t in range(7):
            p = 2 * j + t - 3
            if p < 0:
                p = -p
            elif p > n - 1:
                p = 2 * (n - 1) - p
            a[p, j] += _TAPS[t]
    return a.astype(np.float32)


def _hist_body(x_ref, y_ref, a_ref, sx_ref, sy_ref, *, cb, h):
    m = h // 2
    @pl.when(pl.program_id(1) == 0)
    def _init():
        sx_ref[...] = jnp.zeros_like(sx_ref)
        sy_ref[...] = jnp.zeros_like(sy_ref)

    a = a_ref[...]
    for src, dst in ((x_ref, sx_ref), (y_ref, sy_ref)):
        accs = [jnp.zeros((8, m), jnp.float32) for _ in range(_NBINS - 1)]
        imgs = src[0].reshape(cb * h, h)
        tmp_all = jax.lax.dot_general(
            imgs, a, (((1,), (0,)), ((), ())),
            preferred_element_type=jnp.float32)
        for i in range(cb):
            tmp = tmp_all[i * h:(i + 1) * h]
            blur = jax.lax.dot_general(
                a, tmp, (((0,), (0,)), ((), ())),
                preferred_element_type=jnp.float32)
            p = jnp.floor(blur * jnp.float32(_NBINS))
            p3 = p.reshape(m // 8, 8, m)
            for k in range(_NBINS - 1):
                t = jnp.minimum(p3, jnp.float32(k + 1))
                accs[k] = accs[k] + jnp.sum(t, axis=0)
        dst[0] += jnp.concatenate(accs, axis=0)


def _sum_groups(s2):
    cols = [jnp.sum(s2[:, 8 * k:8 * (k + 1)], axis=1, keepdims=True)
            for k in range(_NBINS - 1)]
    return jnp.concatenate(cols, axis=1)


def _hist_from_s(s, nb, npix):
    zero = jnp.zeros((nb, 1), jnp.int32)
    c = s - jnp.concatenate([zero, s[:, :_NBINS - 2]], axis=1)
    top = jnp.full((nb, 1), npix, jnp.int32)
    cext = jnp.concatenate([top, c, zero], axis=1)
    return (cext[:, :_NBINS] - cext[:, 1:_NBINS + 1]).astype(jnp.float32)


def _loss_body(sx_ref, sy_ref, o_ref, *, nb, nc, npix, shape):
    sxs = _sum_groups(jnp.sum(sx_ref[...].astype(jnp.int32), axis=2))
    sys_ = _sum_groups(jnp.sum(sy_ref[...].astype(jnp.int32), axis=2))
    hx = _hist_from_s(sxs, nb, npix)
    hy = _hist_from_s(sys_, nb, npix)
    inv = jnp.float32(1.0 / shape)
    total = jnp.float32(0.0)
    for e in range(1, nb + 1):
        w = jnp.float32(nc - nb if e == nb else 1.0)
        for b in range(e):
            sx = jnp.sum(hx[b:e, :], axis=0) * inv
            sy = jnp.sum(hy[b:e, :], axis=0) * inv
            dot = jnp.sum(sx * sy)
            nx = jnp.sqrt(jnp.sum(sx * sx))
            ny = jnp.sqrt(jnp.sum(sy * sy))
            cos = dot / jnp.maximum(nx * ny, jnp.float32(_EPS))
            total = total + w * cos
    o_ref[...] = jnp.broadcast_to(total / jnp.float32(nb * nc), (1, 1))


def kernel(x, y):
    b, c, h, w = x.shape
    m = h // 2
    cb = 48
    a = jnp.asarray(_blur_matrix(h))

    sx, sy = pl.pallas_call(
        functools.partial(_hist_body, cb=cb, h=h),
        grid=(b, c // cb),
        in_specs=[
            pl.BlockSpec((1, cb, h, w), lambda i, j: (i, j, 0, 0)),
            pl.BlockSpec((1, cb, h, w), lambda i, j: (i, j, 0, 0)),
            pl.BlockSpec((h, m), lambda i, j: (0, 0)),
        ],
        out_specs=[
            pl.BlockSpec((1, 8 * (_NBINS - 1), m), lambda i, j: (i, 0, 0)),
            pl.BlockSpec((1, 8 * (_NBINS - 1), m), lambda i, j: (i, 0, 0)),
        ],
        out_shape=[
            jax.ShapeDtypeStruct((b, 8 * (_NBINS - 1), m), jnp.float32),
            jax.ShapeDtypeStruct((b, 8 * (_NBINS - 1), m), jnp.float32),
        ],
    )(x, y, a)

    out = pl.pallas_call(
        functools.partial(
            _loss_body, nb=b, nc=c, npix=c * m * m, shape=h * w),
        out_shape=jax.ShapeDtypeStruct((1, 1), jnp.float32),
    )(sx, sy)
    return out[0, 0]


# speedup vs baseline: 1.0470x; 1.0470x over previous
import functools

import numpy as np
import jax
import jax.numpy as jnp
from jax.experimental import pallas as pl
from jax.experimental.pallas import tpu as pltpu

_NBINS = 25
_EPS = 1e-6
_TAPS = np.array([1.0, 6.0, 15.0, 20.0, 15.0, 6.0, 1.0], dtype=np.float64) / 64.0


def _blur_matrix(n: int) -> np.ndarray:
    m = n // 2
    a = np.zeros((n, m), dtype=np.float64)
    for j in range(m):
        for t in range(7):
            p = 2 * j + t - 3
            if p < 0:
                p = -p
            elif p > n - 1:
                p = 2 * (n - 1) - p
            a[p, j] += _TAPS[t]
    return a.astype(np.float32)


def _hist_body(x_ref, y_ref, a_ref, o_ref, sx_scr, sy_scr, *, cb, h, nb, nc):
    m = h // 2
    bi = pl.program_id(0)
    ci = pl.program_id(1)
    @pl.when((bi == 0) & (ci == 0))
    def _init():
        sx_scr[...] = jnp.zeros_like(sx_scr)
        sy_scr[...] = jnp.zeros_like(sy_scr)

    a = a_ref[...]
    for src, dst in ((x_ref, sx_scr), (y_ref, sy_scr)):
        accs = [jnp.zeros((8, m), jnp.float32) for _ in range(_NBINS - 1)]
        imgs = src[0].reshape(cb * h, h)
        tmp_all = jax.lax.dot_general(
            imgs, a, (((1,), (0,)), ((), ())),
            preferred_element_type=jnp.float32)
        for i in range(cb):
            tmp = tmp_all[i * h:(i + 1) * h]
            blur = jax.lax.dot_general(
                a, tmp, (((0,), (0,)), ((), ())),
                preferred_element_type=jnp.float32)
            p = jnp.floor(blur * jnp.float32(_NBINS))
            p3 = p.reshape(m // 8, 8, m)
            for k in range(_NBINS - 1):
                t = jnp.minimum(p3, jnp.float32(k + 1))
                accs[k] = accs[k] + jnp.sum(t, axis=0)
        dst[bi] += jnp.concatenate(accs, axis=0)

    @pl.when((bi == nb - 1) & (ci == pl.num_programs(1) - 1))
    def _finish():
        _loss_from_s(sx_scr[...], sy_scr[...], o_ref,
                     nb=nb, nc=nc, npix=nc * m * m, shape=h * h)


def _sum_groups(s2):
    cols = [jnp.sum(s2[:, 8 * k:8 * (k + 1)], axis=1, keepdims=True)
            for k in range(_NBINS - 1)]
    return jnp.concatenate(cols, axis=1)


def _hist_from_s(s, nb, npix):
    zero = jnp.zeros((nb, 1), jnp.int32)
    c = s - jnp.concatenate([zero, s[:, :_NBINS - 2]], axis=1)
    top = jnp.full((nb, 1), npix, jnp.int32)
    cext = jnp.concatenate([top, c, zero], axis=1)
    return (cext[:, :_NBINS] - cext[:, 1:_NBINS + 1]).astype(jnp.float32)


def _loss_from_s(sx_val, sy_val, o_ref, *, nb, nc, npix, shape):
    sxs = _sum_groups(jnp.sum(sx_val.astype(jnp.int32), axis=2))
    sys_ = _sum_groups(jnp.sum(sy_val.astype(jnp.int32), axis=2))
    hx = _hist_from_s(sxs, nb, npix)
    hy = _hist_from_s(sys_, nb, npix)
    inv = jnp.float32(1.0 / shape)
    total = jnp.float32(0.0)
    for e in range(1, nb + 1):
        w = jnp.float32(nc - nb if e == nb else 1.0)
        for b in range(e):
            sx = jnp.sum(hx[b:e, :], axis=0) * inv
            sy = jnp.sum(hy[b:e, :], axis=0) * inv
            dot = jnp.sum(sx * sy)
            nx = jnp.sqrt(jnp.sum(sx * sx))
            ny = jnp.sqrt(jnp.sum(sy * sy))
            cos = dot / jnp.maximum(nx * ny, jnp.float32(_EPS))
            total = total + w * cos
    o_ref[...] = jnp.broadcast_to(total / jnp.float32(nb * nc), (1, 1))


def kernel(x, y):
    b, c, h, w = x.shape
    m = h // 2
    cb = 32
    a = jnp.asarray(_blur_matrix(h))

    out = pl.pallas_call(
        functools.partial(_hist_body, cb=cb, h=h, nb=b, nc=c),
        grid=(b, c // cb),
        in_specs=[
            pl.BlockSpec((1, cb, h, w), lambda i, j: (i, j, 0, 0)),
            pl.BlockSpec((1, cb, h, w), lambda i, j: (i, j, 0, 0)),
            pl.BlockSpec((h, m), lambda i, j: (0, 0)),
        ],
        out_specs=pl.BlockSpec((1, 1), lambda i, j: (0, 0)),
        out_shape=jax.ShapeDtypeStruct((1, 1), jnp.float32),
        scratch_shapes=[
            pltpu.VMEM((b, 8 * (_NBINS - 1), m), jnp.float32),
            pltpu.VMEM((b, 8 * (_NBINS - 1), m), jnp.float32),
        ],
    )(x, y, a)
    return out[0, 0]
